# baseline (speedup 1.0000x reference)
import functools

import jax
import jax.numpy as jnp
from jax import lax
from jax.experimental import pallas as pl
from jax.experimental.pallas import tpu as pltpu
from jax.experimental.pallas import tpu_sc as plsc

MAX_NORM = 1.0


def _sc_gather(idx_flat, table):
    N = idx_flat.shape[0]
    V, D = table.shape
    info = plsc.get_sparse_core_info()
    nc, ns = info.num_cores, info.num_subcores
    nw = nc * ns
    b_per_w = N // nw
    mesh = plsc.VectorSubcoreMesh(core_axis_name="c", subcore_axis_name="s")

    @functools.partial(
        pl.kernel,
        mesh=mesh,
        compiler_params=pltpu.CompilerParams(use_tc_tiling_on_sc=False),
        out_type=jax.ShapeDtypeStruct((N, D), jnp.float32),
        scratch_types=[
            pltpu.VMEM((b_per_w,), jnp.int32),
            pltpu.VMEM((b_per_w, D), jnp.float32),
            pltpu.SemaphoreType.DMA,
        ],
    )
    def k(table_hbm, idx_hbm, out_hbm, idx_v, rows_v, sem):
        wid = lax.axis_index("s") * nc + lax.axis_index("c")
        base = wid * b_per_w
        pltpu.sync_copy(idx_hbm.at[pl.ds(base, b_per_w)], idx_v)
        pltpu.async_copy(table_hbm.at[idx_v], rows_v, sem).wait()
        pltpu.sync_copy(rows_v, out_hbm.at[pl.ds(base, b_per_w)])

    return k(table, idx_flat)


def _pool(emb):
    B, L, D = emb.shape
    blk = 256

    def body(e_ref, x_ref):
        e = e_ref[...]
        s = jnp.sum(e * e, axis=-1, keepdims=True)
        scale = jnp.minimum(MAX_NORM, 1.0 / jnp.maximum(jnp.sqrt(s), 1e-7))
        x_ref[...] = jnp.mean(e * scale, axis=1)

    return pl.pallas_call(
        body,
        grid=(B // blk,),
        in_specs=[pl.BlockSpec((blk, L, D), lambda i: (i, 0, 0))],
        out_specs=pl.BlockSpec((blk, D), lambda i: (i, 0)),
        out_shape=jax.ShapeDtypeStruct((B, D), jnp.float32),
    )(emb)


def _project(x, W, b2):
    B, D = x.shape
    V = W.shape[0]
    vt = 1024
    nv = pl.cdiv(V, vt)

    def body(x_ref, w_ref, b_ref, o_ref):
        o_ref[...] = lax.dot_general(
            x_ref[...], w_ref[...], (((1,), (1,)), ((), ())),
            preferred_element_type=jnp.float32,
        ) + b_ref[...]

    return pl.pallas_call(
        body,
        grid=(nv,),
        in_specs=[
            pl.BlockSpec((B, D), lambda j: (0, 0)),
            pl.BlockSpec((vt, D), lambda j: (j, 0)),
            pl.BlockSpec((1, vt), lambda j: (0, j)),
        ],
        out_specs=pl.BlockSpec((B, vt), lambda j: (0, j)),
        out_shape=jax.ShapeDtypeStruct((B, V), jnp.float32),
    )(x, W, b2)


def kernel(inputs_, emb_table, W, b):
    B, L = inputs_.shape
    V, D = emb_table.shape
    idx = inputs_.reshape(B * L).astype(jnp.int32)
    rows = _sc_gather(idx, emb_table)
    x = _pool(rows.reshape(B, L, D))
    return _project(x, W, b.reshape(1, V))


# speedup vs baseline: 1.0629x; 1.0629x over previous
import functools

import jax
import jax.numpy as jnp
from jax import lax
from jax.experimental import pallas as pl
from jax.experimental.pallas import tpu as pltpu
from jax.experimental.pallas import tpu_sc as plsc

MAX_NORM = 1.0
LANES = 16


def _vrsqrt(s):
    i = lax.bitcast_convert_type(s, jnp.int32)
    y = lax.bitcast_convert_type(
        jnp.int32(0x5F3759DF) - lax.shift_right_logical(i, 1), jnp.float32)
    half = 0.5 * s
    for _ in range(3):
        y = y * (1.5 - half * y * y)
    return y


def _sc_pool(idx_flat, table, B, L):
    N = idx_flat.shape[0]
    V, D = table.shape
    n_chunk = D // LANES
    info = plsc.get_sparse_core_info()
    nc, ns = info.num_cores, info.num_subcores
    nw = nc * ns
    b_per_w = B // nw
    r_per_w = b_per_w * L
    mesh = plsc.VectorSubcoreMesh(core_axis_name="c", subcore_axis_name="s")

    @functools.partial(
        pl.kernel,
        mesh=mesh,
        compiler_params=pltpu.CompilerParams(
            use_tc_tiling_on_sc=False, needs_layout_passes=False),
        out_type=jax.ShapeDtypeStruct((B, D), jnp.float32),
        scratch_types=[
            pltpu.VMEM((r_per_w,), jnp.int32),
            pltpu.VMEM((r_per_w, D), jnp.float32),
            pltpu.VMEM((b_per_w, D), jnp.float32),
            pltpu.SemaphoreType.DMA,
        ],
    )
    def k(table_hbm, idx_hbm, out_hbm, idx_v, rows_v, x_v, sem):
        wid = lax.axis_index("s") * nc + lax.axis_index("c")
        base = wid * r_per_w
        pltpu.sync_copy(idx_hbm.at[pl.ds(base, r_per_w)], idx_v)
        pltpu.async_copy(table_hbm.at[idx_v], rows_v, sem).wait()

        inv_l = jnp.float32(1.0 / L)

        def batch_body(bi, _):
            def row_body(li, accs):
                r = bi * L + li
                chunks = [rows_v[r, pl.ds(c * LANES, LANES)]
                          for c in range(n_chunk)]
                sq = chunks[0] * chunks[0]
                for c in range(1, n_chunk):
                    sq = sq + chunks[c] * chunks[c]
                s = lax.reduce_sum_p.bind(sq, axes=(0,))
                s_vec = jnp.full((LANES,), s, dtype=jnp.float32)
                scale = jnp.minimum(
                    jnp.float32(MAX_NORM),
                    _vrsqrt(jnp.maximum(s_vec, jnp.float32(1e-14))))
                return tuple(a + chunks[c] * scale for c, a in enumerate(accs))

            accs = lax.fori_loop(
                0, L, row_body,
                tuple(jnp.zeros((LANES,), jnp.float32)
                      for _ in range(n_chunk)))
            for c in range(n_chunk):
                x_v[bi, pl.ds(c * LANES, LANES)] = accs[c] * inv_l
            return 0

        lax.fori_loop(0, b_per_w, batch_body, 0)
        pltpu.sync_copy(x_v, out_hbm.at[pl.ds(wid * b_per_w, b_per_w)])

    return k(table, idx_flat)


def _project(x, W, b2):
    B, D = x.shape
    V = W.shape[0]
    vt = 1024
    nv = pl.cdiv(V, vt)

    def body(x_ref, w_ref, b_ref, o_ref):
        o_ref[...] = lax.dot_general(
            x_ref[...], w_ref[...], (((1,), (1,)), ((), ())),
            preferred_element_type=jnp.float32,
        ) + b_ref[...]

    return pl.pallas_call(
        body,
        grid=(nv,),
        in_specs=[
            pl.BlockSpec((B, D), lambda j: (0, 0)),
            pl.BlockSpec((vt, D), lambda j: (j, 0)),
            pl.BlockSpec((1, vt), lambda j: (0, j)),
        ],
        out_specs=pl.BlockSpec((B, vt), lambda j: (0, j)),
        out_shape=jax.ShapeDtypeStruct((B, V), jnp.float32),
    )(x, W, b2)


def kernel(inputs_, emb_table, W, b):
    B, L = inputs_.shape
    V, D = emb_table.shape
    idx = inputs_.reshape(B * L).astype(jnp.int32)
    x = _sc_pool(idx, emb_table, B, L)
    return _project(x, W, b.reshape(1, V))


# speedup vs baseline: 2.0094x; 1.8905x over previous
import functools

import jax
import jax.numpy as jnp
from jax import lax
from jax.experimental import pallas as pl
from jax.experimental.pallas import tpu as pltpu
from jax.experimental.pallas import tpu_sc as plsc

MAX_NORM = 1.0
LANES = 16


def _vrsqrt(s):
    i = lax.bitcast_convert_type(s, jnp.int32)
    y = lax.bitcast_convert_type(
        jnp.int32(0x5F3759DF) - lax.shift_right_logical(i, 1), jnp.float32)
    half = 0.5 * s
    for _ in range(3):
        y = y * (1.5 - half * y * y)
    return y


def _sc_pool(idx_flat, table, B, L):
    N = idx_flat.shape[0]
    V, D = table.shape
    n_chunk = D // LANES
    info = plsc.get_sparse_core_info()
    nc, ns = info.num_cores, info.num_subcores
    nw = nc * ns
    b_per_w = B // nw
    r_per_w = b_per_w * L
    mesh = plsc.VectorSubcoreMesh(core_axis_name="c", subcore_axis_name="s")

    @functools.partial(
        pl.kernel,
        mesh=mesh,
        compiler_params=pltpu.CompilerParams(
            use_tc_tiling_on_sc=False, needs_layout_passes=False),
        out_type=jax.ShapeDtypeStruct((B, D), jnp.float32),
        scratch_types=[
            pltpu.VMEM((r_per_w,), jnp.int32),
            pltpu.VMEM((r_per_w, D), jnp.float32),
            pltpu.VMEM((b_per_w, D), jnp.float32),
            pltpu.SemaphoreType.DMA,
        ],
    )
    def k(table_hbm, idx_hbm, out_hbm, idx_v, rows_v, x_v, sem):
        wid = lax.axis_index("s") * nc + lax.axis_index("c")
        base = wid * r_per_w
        pltpu.sync_copy(idx_hbm.at[pl.ds(base, r_per_w)], idx_v)
        pltpu.async_copy(table_hbm.at[idx_v], rows_v, sem).wait()

        inv_l = jnp.float32(1.0 / L)

        def batch_body(bi, _):
            def row_body(li, accs):
                r = bi * L + li
                chunks = [rows_v[r, pl.ds(c * LANES, LANES)]
                          for c in range(n_chunk)]
                sq = chunks[0] * chunks[0]
                for c in range(1, n_chunk):
                    sq = sq + chunks[c] * chunks[c]
                s = lax.reduce_sum_p.bind(sq, axes=(0,))
                s_vec = jnp.full((LANES,), s, dtype=jnp.float32)
                scale = jnp.minimum(
                    jnp.float32(MAX_NORM),
                    _vrsqrt(jnp.maximum(s_vec, jnp.float32(1e-14))))
                return tuple(a + chunks[c] * scale for c, a in enumerate(accs))

            accs = lax.fori_loop(
                0, L, row_body,
                tuple(jnp.zeros((LANES,), jnp.float32)
                      for _ in range(n_chunk)))
            for c in range(n_chunk):
                x_v[bi, pl.ds(c * LANES, LANES)] = accs[c] * inv_l
            return 0

        lax.fori_loop(0, b_per_w, batch_body, 0)
        pltpu.sync_copy(x_v, out_hbm.at[pl.ds(wid * b_per_w, b_per_w)])

    return k(table, idx_flat)


def _project(x, W, b2):
    B, D = x.shape
    V = W.shape[0]
    vt = 1024
    nv = pl.cdiv(V, vt)

    def body(w_ref, x_ref, b_ref, o_ref):
        o_ref[...] = lax.dot_general(
            w_ref[...], x_ref[...], (((1,), (1,)), ((), ())),
            preferred_element_type=jnp.float32,
        ) + b_ref[...]

    return pl.pallas_call(
        body,
        grid=(nv,),
        in_specs=[
            pl.BlockSpec((vt, D), lambda j: (j, 0)),
            pl.BlockSpec((B, D), lambda j: (0, 0)),
            pl.BlockSpec((vt, 1), lambda j: (j, 0)),
        ],
        out_specs=pl.BlockSpec((vt, B), lambda j: (j, 0)),
        out_shape=jax.ShapeDtypeStruct((V, B), jnp.float32),
    )(W, x, b2)


def kernel(inputs_, emb_table, W, b):
    B, L = inputs_.shape
    V, D = emb_table.shape
    idx = inputs_.reshape(B * L).astype(jnp.int32)
    x = _sc_pool(idx, emb_table, B, L)
    return _project(x, W, b.reshape(V, 1)).T


# speedup vs baseline: 2.6033x; 1.2956x over previous
import functools

import jax
import jax.numpy as jnp
from jax import lax
from jax.experimental import pallas as pl
from jax.experimental.pallas import tpu as pltpu
from jax.experimental.pallas import tpu_sc as plsc

MAX_NORM = 1.0
LANES = 16


def _vrsqrt(s):
    i = lax.bitcast_convert_type(s, jnp.int32)
    y = lax.bitcast_convert_type(
        jnp.int32(0x5F3759DF) - lax.shift_right_logical(i, 1), jnp.float32)
    half = 0.5 * s
    for _ in range(3):
        y = y * (1.5 - half * y * y)
    return y


def _sc_pool(idx_flat, table, B, L):
    N = idx_flat.shape[0]
    V, D = table.shape
    n_chunk = D // LANES
    info = plsc.get_sparse_core_info()
    nc, ns = info.num_cores, info.num_subcores
    nw = nc * ns
    b_per_w = B // nw
    r_per_w = b_per_w * L
    mesh = plsc.VectorSubcoreMesh(core_axis_name="c", subcore_axis_name="s")

    @functools.partial(
        pl.kernel,
        mesh=mesh,
        compiler_params=pltpu.CompilerParams(
            use_tc_tiling_on_sc=False, needs_layout_passes=False),
        out_type=jax.ShapeDtypeStruct((B, D), jnp.float32),
        scratch_types=[
            pltpu.VMEM((r_per_w,), jnp.int32),
            pltpu.VMEM((r_per_w, D), jnp.float32),
            pltpu.VMEM((b_per_w, D), jnp.float32),
            pltpu.SemaphoreType.DMA,
        ],
    )
    def k(table_hbm, idx_hbm, out_hbm, idx_v, rows_v, x_v, sem):
        wid = lax.axis_index("s") * nc + lax.axis_index("c")
        base = wid * r_per_w
        pltpu.sync_copy(idx_hbm.at[pl.ds(base, r_per_w)], idx_v)
        pltpu.async_copy(table_hbm.at[idx_v], rows_v, sem).wait()

        inv_l = jnp.float32(1.0 / L)

        def batch_body(bi, _):
            def row_body(li, accs):
                r = bi * L + li
                chunks = [rows_v[r, pl.ds(c * LANES, LANES)]
                          for c in range(n_chunk)]
                sq = chunks[0] * chunks[0]
                for c in range(1, n_chunk):
                    sq = sq + chunks[c] * chunks[c]
                s = lax.reduce_sum_p.bind(sq, axes=(0,))
                s_vec = jnp.full((LANES,), s, dtype=jnp.float32)
                scale = jnp.minimum(
                    jnp.float32(MAX_NORM),
                    _vrsqrt(jnp.maximum(s_vec, jnp.float32(1e-14))))
                return tuple(a + chunks[c] * scale for c, a in enumerate(accs))

            accs = lax.fori_loop(
                0, L, row_body,
                tuple(jnp.zeros((LANES,), jnp.float32)
                      for _ in range(n_chunk)))
            for c in range(n_chunk):
                x_v[bi, pl.ds(c * LANES, LANES)] = accs[c] * inv_l
            return 0

        lax.fori_loop(0, b_per_w, batch_body, 0)
        pltpu.sync_copy(x_v, out_hbm.at[pl.ds(wid * b_per_w, b_per_w)])

    return k(table, idx_flat)


def _project(x, Wt, b2):
    B, D = x.shape
    V = Wt.shape[1]
    vt = 1024
    nv = pl.cdiv(V, vt)

    def body(w_ref, x_ref, b_ref, o_ref):
        acc = lax.dot_general(
            w_ref[...], x_ref[...], (((0,), (1,)), ((), ())),
            preferred_element_type=jnp.float32,
        )
        ones = jnp.ones((B, 1), jnp.float32)
        bias = lax.dot_general(
            b_ref[...], ones, (((0,), (1,)), ((), ())),
            preferred_element_type=jnp.float32,
        )
        o_ref[...] = acc + bias

    return pl.pallas_call(
        body,
        grid=(nv,),
        in_specs=[
            pl.BlockSpec((D, vt), lambda j: (0, j)),
            pl.BlockSpec((B, D), lambda j: (0, 0)),
            pl.BlockSpec((1, vt), lambda j: (0, j)),
        ],
        out_specs=pl.BlockSpec((vt, B), lambda j: (j, 0)),
        out_shape=jax.ShapeDtypeStruct((V, B), jnp.float32),
    )(Wt, x, b2)


def kernel(inputs_, emb_table, W, b):
    B, L = inputs_.shape
    V, D = emb_table.shape
    idx = inputs_.reshape(B * L).astype(jnp.int32)
    x = _sc_pool(idx, emb_table, B, L)
    return _project(x, W.T, b.reshape(1, V)).T


# speedup vs baseline: 2.8910x; 1.1105x over previous
import functools

import jax
import jax.numpy as jnp
from jax import lax
from jax.experimental import pallas as pl
from jax.experimental.pallas import tpu as pltpu
from jax.experimental.pallas import tpu_sc as plsc

MAX_NORM = 1.0
LANES = 16


def _vrsqrt(s):
    i = lax.bitcast_convert_type(s, jnp.int32)
    y = lax.bitcast_convert_type(
        jnp.int32(0x5F3759DF) - lax.shift_right_logical(i, 1), jnp.float32)
    half = 0.5 * s
    for _ in range(3):
        y = y * (1.5 - half * y * y)
    return y


def _sc_pool(idx_flat, table, B, L):
    N = idx_flat.shape[0]
    V, D = table.shape
    n_chunk = D // LANES
    info = plsc.get_sparse_core_info()
    nc, ns = info.num_cores, info.num_subcores
    nw = nc * ns
    b_per_w = B // nw
    r_per_w = b_per_w * L
    mesh = plsc.VectorSubcoreMesh(core_axis_name="c", subcore_axis_name="s")

    @functools.partial(
        pl.kernel,
        mesh=mesh,
        compiler_params=pltpu.CompilerParams(
            use_tc_tiling_on_sc=False, needs_layout_passes=False),
        out_type=jax.ShapeDtypeStruct((B, D), jnp.float32),
        scratch_types=[
            pltpu.VMEM((r_per_w,), jnp.int32),
            pltpu.VMEM((r_per_w, D), jnp.float32),
            pltpu.VMEM((b_per_w, D), jnp.float32),
            pltpu.SemaphoreType.DMA,
        ],
    )
    def k(table_hbm, idx_hbm, out_hbm, idx_v, rows_v, x_v, sem):
        wid = lax.axis_index("s") * nc + lax.axis_index("c")
        base = wid * r_per_w
        pltpu.sync_copy(idx_hbm.at[pl.ds(base, r_per_w)], idx_v)
        pltpu.async_copy(table_hbm.at[idx_v], rows_v, sem).wait()

        inv_l = jnp.float32(1.0 / L)

        def batch_body(bi, _):
            def row_body(li, accs):
                r = bi * L + li
                chunks = [rows_v[r, pl.ds(c * LANES, LANES)]
                          for c in range(n_chunk)]
                sq = chunks[0] * chunks[0]
                for c in range(1, n_chunk):
                    sq = sq + chunks[c] * chunks[c]
                s = lax.reduce_sum_p.bind(sq, axes=(0,))
                s_vec = jnp.full((LANES,), s, dtype=jnp.float32)
                scale = jnp.minimum(
                    jnp.float32(MAX_NORM),
                    _vrsqrt(jnp.maximum(s_vec, jnp.float32(1e-14))))
                return tuple(a + chunks[c] * scale for c, a in enumerate(accs))

            accs = lax.fori_loop(
                0, L, row_body,
                tuple(jnp.zeros((LANES,), jnp.float32)
                      for _ in range(n_chunk)))
            for c in range(n_chunk):
                x_v[bi, pl.ds(c * LANES, LANES)] = accs[c] * inv_l
            return 0

        lax.fori_loop(0, b_per_w, batch_body, 0)
        pltpu.sync_copy(x_v, out_hbm.at[pl.ds(wid * b_per_w, b_per_w)])

    return k(table, idx_flat)


def _project(x, Wt, b2):
    B, D = x.shape
    V = Wt.shape[1]
    vt = 2048
    nv = pl.cdiv(V, vt)

    def body(w_ref, x_ref, b_ref, o_ref):
        acc = lax.dot_general(
            w_ref[...], x_ref[...], (((0,), (1,)), ((), ())),
            preferred_element_type=jnp.float32,
        )
        ones = jnp.ones((B, 1), jnp.float32)
        bias = lax.dot_general(
            b_ref[...], ones, (((0,), (1,)), ((), ())),
            preferred_element_type=jnp.float32,
        )
        o_ref[...] = acc + bias

    return pl.pallas_call(
        body,
        grid=(nv,),
        in_specs=[
            pl.BlockSpec((D, vt), lambda j: (0, j)),
            pl.BlockSpec((B, D), lambda j: (0, 0)),
            pl.BlockSpec((1, vt), lambda j: (0, j)),
        ],
        out_specs=pl.BlockSpec((vt, B), lambda j: (j, 0)),
        out_shape=jax.ShapeDtypeStruct((V, B), jnp.float32),
    )(Wt, x, b2)


def kernel(inputs_, emb_table, W, b):
    B, L = inputs_.shape
    V, D = emb_table.shape
    idx = inputs_.reshape(B * L).astype(jnp.int32)
    x = _sc_pool(idx, emb_table, B, L)
    return _project(x, W.T, b.reshape(1, V)).T
